# baseline (device time: 861398 ns/iter reference)
import jax
import jax.numpy as jnp
from jax import lax
from jax.experimental import pallas as pl
from jax.experimental.pallas import tpu as pltpu

N_DEV = 16
CHUNK = 256
N_LANES = 4

RING = [0, 4, 8, 12, 13, 9, 5, 1, 2, 6, 10, 14, 15, 11, 7, 3]
POS = [RING.index(p) for p in range(N_DEV)]
SUCC = [0] * N_DEV
PRED = [0] * N_DEV
for _a, _p in enumerate(RING):
    SUCC[_p] = RING[(_a + 1) % N_DEV]
    PRED[_p] = RING[(_a - 1) % N_DEV]


def _lut(idx, table):
    v = jnp.int32(table[0])
    for k in range(1, N_DEV):
        v = jnp.where(idx == k, jnp.int32(table[k]), v)
    return v


class _Lane:

    def __init__(self, to_id, from_id, col0, sgn, send, recv, stage, own,
                 send_sems, recv_sems, out_sems, own_sem, credit):
        self.to_id = to_id
        self.from_id = from_id
        self.col0 = col0
        self.sgn = sgn
        self.send = send
        self.recv = recv
        self.stage = stage
        self.own = own
        self.send_sems = send_sems
        self.recv_sems = recv_sems
        self.out_sems = out_sems
        self.own_sem = own_sem
        self.credit = credit
        self.copies = {}
        self.prev_rdma = None


def kernel(x, w_mat, scale_x, scale_w):
    m, k_loc = x.shape
    _, n = w_mat.shape
    nl = n // N_LANES
    assert m == N_DEV * CHUNK

    def body(x_ref, w_ref, sx_ref, sw_ref, out_ref, wb_ref, *rest):
        p = lax.axis_index("i")
        j = _lut(p, POS)
        succ = _lut(p, SUCC)
        pred = _lut(p, PRED)

        bufs, sems = rest[:4 * N_LANES], rest[4 * N_LANES:]
        lanes = []
        for k in range(N_LANES):
            cw = k < N_LANES // 2
            lanes.append(_Lane(
                succ if cw else pred,
                pred if cw else succ,
                k * nl,
                -1 if cw else 1,
                *bufs[4 * k:4 * k + 4],
                *sems[5 * k:5 * k + 5],
            ))

        barrier = pltpu.get_barrier_semaphore()
        for nbr in (succ, pred):
            pl.semaphore_signal(barrier, inc=1, device_id=(nbr,),
                                device_id_type=pl.DeviceIdType.MESH)
        pl.semaphore_wait(barrier, 2)

        wb_ref[...] = w_ref[...].astype(jnp.bfloat16)

        def partial(c, col0):
            xc = x_ref[pl.ds(c * CHUNK, CHUNK), :].astype(jnp.bfloat16)
            return jnp.dot(xc, wb_ref[:, pl.ds(col0, nl)],
                           preferred_element_type=jnp.float32)

        for d in lanes:
            d.send[0] = partial(j, d.col0).astype(jnp.bfloat16)

        scale = sx_ref[0] * sw_ref[0]

        for t in range(N_DEV - 1):
            slot = t % 2
            rdmas = []
            for d in lanes:
                if t >= 2:
                    pl.semaphore_wait(d.credit, 1)
                r = pltpu.make_async_remote_copy(
                    src_ref=d.send.at[slot],
                    dst_ref=d.recv.at[slot],
                    send_sem=d.send_sems.at[slot],
                    recv_sem=d.recv_sems.at[slot],
                    device_id=(d.to_id,),
                    device_id_type=pl.DeviceIdType.MESH,
                )
                r.start()
                rdmas.append(r)
            pcs = [partial((j + d.sgn * (t + 1)) % N_DEV, d.col0)
                   for d in lanes]
            for d, r, pc in zip(lanes, rdmas, pcs):
                r.wait_recv()
                if t >= 1:
                    d.prev_rdma.wait_send()
                acc = pc + d.recv[slot].astype(jnp.float32)
                if t < N_DEV - 2:
                    d.send[(t + 1) % 2] = acc.astype(jnp.bfloat16)
                else:
                    fin = jnp.maximum(acc * scale, 0.0)
                    d.own[...] = fin
                    d.send[1] = fin.astype(jnp.bfloat16)
                pl.semaphore_signal(d.credit, inc=1, device_id=(d.from_id,),
                                    device_id_type=pl.DeviceIdType.MESH)
                d.prev_rdma = r

        own_copies = []
        for d in lanes:
            oc = (j - d.sgn) % N_DEV
            cp = pltpu.make_async_copy(
                d.own,
                out_ref.at[pl.ds(oc * CHUNK, CHUNK), pl.ds(d.col0, nl)],
                d.own_sem)
            cp.start()
            own_copies.append(cp)

        for d in lanes:
            d.prev_rdma.wait_send()

        def convert_and_store(d, s):
            pslot = s % 2
            if (s - 2) in d.copies:
                d.copies[s - 2].wait()
            d.stage[pslot] = d.recv[pslot].astype(jnp.float32)
            rc = (j + d.sgn * (s - (N_DEV - 1))) % N_DEV
            cp = pltpu.make_async_copy(
                d.stage.at[pslot],
                out_ref.at[pl.ds(rc * CHUNK, CHUNK), pl.ds(d.col0, nl)],
                d.out_sems.at[pslot])
            cp.start()
            d.copies[s] = cp

        for t in range(N_DEV - 1, 2 * N_DEV - 2):
            slot = t % 2
            rdmas = []
            for d in lanes:
                pl.semaphore_wait(d.credit, 1)
                if t == N_DEV - 1:
                    src = d.send.at[1]
                else:
                    src = d.recv.at[(t + 1) % 2]
                r = pltpu.make_async_remote_copy(
                    src_ref=src,
                    dst_ref=d.recv.at[slot],
                    send_sem=d.send_sems.at[slot],
                    recv_sem=d.recv_sems.at[slot],
                    device_id=(d.to_id,),
                    device_id_type=pl.DeviceIdType.MESH,
                )
                r.start()
                rdmas.append(r)
            if t >= N_DEV:
                for d in lanes:
                    convert_and_store(d, t - 1)
            for d, r in zip(lanes, rdmas):
                r.wait_recv()
            for d, r in zip(lanes, rdmas):
                r.wait_send()
                if N_DEV <= t <= 2 * N_DEV - 4:
                    pl.semaphore_signal(
                        d.credit, inc=1, device_id=(d.from_id,),
                        device_id_type=pl.DeviceIdType.MESH)

        for d in lanes:
            convert_and_store(d, 2 * N_DEV - 3)
            d.copies[2 * N_DEV - 4].wait()
            d.copies[2 * N_DEV - 3].wait()
        for cp in own_copies:
            cp.wait()

    lane_bufs = []
    lane_sems = []
    for _ in range(N_LANES):
        lane_bufs += [
            pltpu.VMEM((2, CHUNK, nl), jnp.bfloat16),
            pltpu.VMEM((2, CHUNK, nl), jnp.bfloat16),
            pltpu.VMEM((2, CHUNK, nl), jnp.float32),
            pltpu.VMEM((CHUNK, nl), jnp.float32),
        ]
        lane_sems += [
            pltpu.SemaphoreType.DMA((2,)),
            pltpu.SemaphoreType.DMA((2,)),
            pltpu.SemaphoreType.DMA((2,)),
            pltpu.SemaphoreType.DMA,
            pltpu.SemaphoreType.REGULAR,
        ]

    return pl.pallas_call(
        body,
        out_shape=jax.ShapeDtypeStruct((m, n), jnp.float32),
        in_specs=[
            pl.BlockSpec(memory_space=pltpu.VMEM),
            pl.BlockSpec(memory_space=pltpu.VMEM),
            pl.BlockSpec(memory_space=pltpu.SMEM),
            pl.BlockSpec(memory_space=pltpu.SMEM),
        ],
        out_specs=pl.BlockSpec(memory_space=pl.ANY),
        scratch_shapes=(
            [pltpu.VMEM((k_loc, n), jnp.bfloat16)] + lane_bufs + lane_sems
        ),
        compiler_params=pltpu.CompilerParams(
            collective_id=0,
            vmem_limit_bytes=110 * 1024 * 1024,
        ),
    )(x, w_mat, scale_x, scale_w)
